# baseline (speedup 1.0000x reference)
import functools

import jax
import jax.numpy as jnp
from jax import lax
from jax.experimental import pallas as pl
from jax.experimental.pallas import tpu as pltpu
from jax.experimental.pallas import tpu_sc as plsc

MAXN = 512
E = MAXN * MAXN
BATCH = 16
NC, NS, LANES = 2, 16, 16
NW = NC * NS
K = 4096
KSH = 12
TPG = E // K
NPAIR = TPG // 4
NVEC = K // LANES

_MESH = plsc.VectorSubcoreMesh(core_axis_name="c", subcore_axis_name="s")


def _bcast_gather(v, idx):
    return lax.gather(
        v, idx[:, None],
        dimension_numbers=lax.GatherDimensionNumbers(
            offset_dims=(), collapsed_slice_dims=(0,), start_index_map=(0,)),
        slice_sizes=(1,), mode=lax.GatherScatterMode.PROMISE_IN_BOUNDS)


def _edges_kernel(x_hbm, send_hbm, recv_hbm, nedge_hbm, nnode_hbm,
                  xv, sbufA, rbufA, msbufA, mrbufA,
                  sbufB, rbufB, msbufB, mrbufB,
                  nnbuf, nebuf, semA, semMA, semB, semMB):
    w = lax.axis_index("s") * NC + lax.axis_index("c")

    pltpu.sync_copy(x_hbm, xv)

    lane = lax.iota(jnp.int32, LANES)
    rlane = (LANES - 1) - lane
    v0 = xv[pl.ds(0, LANES)]
    v1 = xv[pl.ds(LANES, LANES)]
    idx2 = (lane << 1) & 15
    n_all = jnp.where(lane < 8, _bcast_gather(v0, idx2), _bcast_gather(v1, idx2))

    half = jnp.float32(0.5)
    three_half = jnp.float32(1.5)
    magic = jnp.full((LANES,), 0x5F3759DF, dtype=jnp.int32)

    def task_id(k2):
        return (k2 << 6) | ((w + 5 * k2) & (NW - 1))

    def classify(t):
        g = t >> 6
        c = t & (TPG - 1)
        base = c * K
        gv = jnp.full((LANES,), g, dtype=jnp.int32)
        n = _bcast_gather(n_all, gv)
        nsq_s = jnp.max(n * n)
        f_k = ((nsq_s >> 1) >> KSH) << KSH
        rho = nsq_s & 7
        paired = base + K <= f_k
        fill = base >= nsq_s
        skip = jnp.logical_and(
            jnp.logical_not(paired),
            jnp.logical_and(base >= nsq_s - f_k - rho,
                            base + K <= nsq_s - rho))
        return g, c, base, gv, n, nsq_s, rho, paired, skip, fill

    def sqrt_d_o(jj):
        a = (jj << 3) + 1
        af = a.astype(jnp.float32)
        y = lax.bitcast_convert_type(
            magic - (lax.bitcast_convert_type(af, jnp.int32) >> 1),
            jnp.float32)
        y = y * (three_half - half * af * y * y)
        y = y * (three_half - half * af * y * y)
        sq = af * y
        d0 = ((sq - 1.0) * half).astype(jnp.int32)
        d0p1 = d0 + 1
        t1 = (d0p1 * (d0 + 2)) >> 1
        up = jj >= t1
        d1 = jnp.where(up, d0p1, d0)
        tt = jnp.where(up, t1, t1 - d0p1)
        dn = jj < tt
        d = jnp.where(dn, d1 - 1, d1)
        tt = jnp.where(dn, tt - d1, tt)
        o = jj - tt
        return d, o

    def do_task(t, sbuf, rbuf, msbuf, mrbuf, sem, semM):
        g, c, base, gv, n, nsq_s, rho, paired, skip, fill = classify(t)
        nm1 = n - 1
        nsq1 = n * n - 1
        h2 = (n * n + 1) >> 1
        goff = gv << 9
        nm1g = nm1 + goff
        general = jnp.logical_not(
            jnp.logical_or(paired, jnp.logical_or(skip, fill)))
        ob = (t >> 6) * E + base

        @pl.when(paired)
        def _():
            idxr = (rho - 1 - lane) & 15
            selr = lane < rho

            def mirror_vals(o, dmo):
                return nm1g - o, nm1g - dmo

            def vec_direct(i):
                jj = jnp.full((LANES,), base + i * LANES, dtype=jnp.int32) + lane
                d, o = sqrt_d_o(jj)
                dmo = d - o
                sbuf[pl.ds(i * LANES, LANES)] = o + goff
                rbuf[pl.ds(i * LANES, LANES)] = dmo + goff
                return mirror_vals(o, dmo)

            def vec_body(i, carry):
                smp, rmp = carry
                smc, rmc = vec_direct(i)
                ri = (NVEC - i) * LANES
                msbuf[pl.ds(ri, LANES)] = jnp.where(
                    selr, _bcast_gather(smc, idxr), _bcast_gather(smp, idxr))
                mrbuf[pl.ds(ri, LANES)] = jnp.where(
                    selr, _bcast_gather(rmc, idxr), _bcast_gather(rmp, idxr))
                return smc, rmc

            sm0, rm0 = vec_direct(0)
            smL, rmL = lax.fori_loop(1, NVEC, vec_body, (sm0, rm0), unroll=16)

            jx = jnp.full((LANES,), base + K, dtype=jnp.int32) + lane
            mx = jx >= h2
            jjx = jnp.where(mx, nsq1 - jx, jx)
            dx, ox = sqrt_d_o(jjx)
            sx = jnp.where(mx, nm1 - ox, ox)
            rx = jnp.where(mx, nm1 - dx + ox, dx - ox)
            smE = nm1g - sx
            rmE = nm1g - rx
            msbuf[pl.ds(0, LANES)] = jnp.where(
                selr, _bcast_gather(smE, idxr), _bcast_gather(smL, idxr))
            mrbuf[pl.ds(0, LANES)] = jnp.where(
                selr, _bcast_gather(rmE, idxr), _bcast_gather(rmL, idxr))

            obm = (t >> 6) * E + (nsq_s >> 3) * 8 - (base + K)
            pltpu.async_copy(sbuf, send_hbm.at[pl.ds(ob, K)], sem)
            pltpu.async_copy(rbuf, recv_hbm.at[pl.ds(ob, K)], sem)
            pltpu.async_copy(msbuf, send_hbm.at[pl.ds(obm, K)], semM)
            pltpu.async_copy(mrbuf, recv_hbm.at[pl.ds(obm, K)], semM)

        @pl.when(general)
        def _():
            def vec_body(i, _):
                j = jnp.full((LANES,), base + i * LANES, dtype=jnp.int32) + lane
                m = j >= h2
                pad = j >= n * n
                jj = jnp.where(m, nsq1 - j, j)
                d, o = sqrt_d_o(jj)
                s = jnp.where(m, nm1 - o, o)
                r = jnp.where(m, nm1 - d + o, d - o)
                sbuf[pl.ds(i * LANES, LANES)] = jnp.where(pad, n, s) + goff
                rbuf[pl.ds(i * LANES, LANES)] = jnp.where(pad, n, r) + goff
                return 0

            lax.fori_loop(0, NVEC, vec_body, 0, unroll=16)
            pltpu.async_copy(sbuf, send_hbm.at[pl.ds(ob, K)], sem)
            pltpu.async_copy(rbuf, recv_hbm.at[pl.ds(ob, K)], sem)

        @pl.when(fill)
        def _():
            fillv = n + goff

            def fill_body(i, _):
                sbuf[pl.ds(i * LANES, LANES)] = fillv
                return 0

            lax.fori_loop(0, NVEC, fill_body, 0, unroll=16)
            pltpu.async_copy(sbuf, send_hbm.at[pl.ds(ob, K)], sem)
            pltpu.async_copy(sbuf, recv_hbm.at[pl.ds(ob, K)], sem)

    def wait_task(t, sbuf, rbuf, msbuf, mrbuf, sem, semM):
        _, _, _, _, _, _, _, paired, skip, _ = classify(t)

        @pl.when(jnp.logical_not(skip))
        def _():
            pltpu.make_async_copy(sbuf, send_hbm.at[pl.ds(0, K)], sem).wait()
            pltpu.make_async_copy(rbuf, recv_hbm.at[pl.ds(0, K)], sem).wait()

        @pl.when(paired)
        def _():
            pltpu.make_async_copy(msbuf, send_hbm.at[pl.ds(0, K)], semM).wait()
            pltpu.make_async_copy(mrbuf, recv_hbm.at[pl.ds(0, K)], semM).wait()

    def pair_body(k2, _):
        tA = task_id(k2)

        @pl.when(k2 > 0)
        def _():
            tP = task_id(k2 - 1)
            wait_task(tP, sbufA, rbufA, msbufA, mrbufA, semA, semMA)

        do_task(tA, sbufA, rbufA, msbufA, mrbufA, semA, semMA)

        @pl.when(k2 > 0)
        def _():
            tP = task_id(k2 - 1) + NW
            wait_task(tP, sbufB, rbufB, msbufB, mrbufB, semB, semMB)

        do_task(tA + NW, sbufB, rbufB, msbufB, mrbufB, semB, semMB)
        return 0

    lax.fori_loop(0, NPAIR, pair_body, 0)

    tL = task_id(NPAIR - 1)
    wait_task(tL, sbufA, rbufA, msbufA, mrbufA, semA, semMA)
    wait_task(tL + NW, sbufB, rbufB, msbufB, mrbufB, semB, semMB)

    @pl.when(w < 2)
    def _():
        vw = jnp.where(jnp.full((LANES,), w, dtype=jnp.int32) == 0, v0, v1)
        nv = _bcast_gather(vw, (lane >> 1) << 1)
        even = (lane & 1) == 0
        nnbuf[...] = jnp.where(even, nv, MAXN - nv)
        nebuf[...] = jnp.where(even, nv * nv, E - nv * nv)
        pltpu.sync_copy(nnbuf, nnode_hbm.at[pl.ds(w * LANES, LANES)])
        pltpu.sync_copy(nebuf, nedge_hbm.at[pl.ds(w * LANES, LANES)])


@jax.jit
def kernel(x):
    out_type = (
        jax.ShapeDtypeStruct((BATCH * E,), jnp.int32),
        jax.ShapeDtypeStruct((BATCH * E,), jnp.int32),
        jax.ShapeDtypeStruct((2 * BATCH,), jnp.int32),
        jax.ShapeDtypeStruct((2 * BATCH,), jnp.int32),
    )
    f = pl.kernel(
        _edges_kernel,
        out_type=out_type,
        mesh=_MESH,
        compiler_params=pltpu.CompilerParams(needs_layout_passes=False),
        scratch_types=[
            pltpu.VMEM((2 * BATCH,), jnp.int32),
            pltpu.VMEM((K,), jnp.int32),
            pltpu.VMEM((K,), jnp.int32),
            pltpu.VMEM((K,), jnp.int32),
            pltpu.VMEM((K,), jnp.int32),
            pltpu.VMEM((K,), jnp.int32),
            pltpu.VMEM((K,), jnp.int32),
            pltpu.VMEM((K,), jnp.int32),
            pltpu.VMEM((K,), jnp.int32),
            pltpu.VMEM((LANES,), jnp.int32),
            pltpu.VMEM((LANES,), jnp.int32),
            pltpu.SemaphoreType.DMA,
            pltpu.SemaphoreType.DMA,
            pltpu.SemaphoreType.DMA,
            pltpu.SemaphoreType.DMA,
        ],
    )
    return f(x.astype(jnp.int32).reshape(2 * BATCH))


# speedup vs baseline: 1.1443x; 1.1443x over previous
import functools

import jax
import jax.numpy as jnp
from jax import lax
from jax.experimental import pallas as pl
from jax.experimental.pallas import tpu as pltpu
from jax.experimental.pallas import tpu_sc as plsc

MAXN = 512
E = MAXN * MAXN
BATCH = 16
NC, NS, LANES = 2, 16, 16
NW = NC * NS
K = 4096
KSH = 12
TPG = E // K
NPAIR = TPG // 4
NVEC = K // LANES

_MESH = plsc.VectorSubcoreMesh(core_axis_name="c", subcore_axis_name="s")


def _bcast_gather(v, idx):
    return lax.gather(
        v, idx[:, None],
        dimension_numbers=lax.GatherDimensionNumbers(
            offset_dims=(), collapsed_slice_dims=(0,), start_index_map=(0,)),
        slice_sizes=(1,), mode=lax.GatherScatterMode.PROMISE_IN_BOUNDS)


def _edges_kernel(x_hbm, send_hbm, recv_hbm, nedge_hbm, nnode_hbm,
                  xv, sbufA, rbufA, msbufA, mrbufA,
                  sbufB, rbufB, msbufB, mrbufB,
                  nnbuf, nebuf, semA, semMA, semB, semMB):
    w = lax.axis_index("s") * NC + lax.axis_index("c")

    pltpu.sync_copy(x_hbm, xv)

    lane = lax.iota(jnp.int32, LANES)
    rlane = (LANES - 1) - lane
    v0 = xv[pl.ds(0, LANES)]
    v1 = xv[pl.ds(LANES, LANES)]
    idx2 = (lane << 1) & 15
    n_all = jnp.where(lane < 8, _bcast_gather(v0, idx2), _bcast_gather(v1, idx2))

    half = jnp.float32(0.5)
    three_half = jnp.float32(1.5)
    magic = jnp.full((LANES,), 0x5F3759DF, dtype=jnp.int32)

    def task_id(k2):
        return (k2 << 6) | ((w + 5 * k2) & (NW - 1))

    def classify(t):
        g = t >> 6
        c = t & (TPG - 1)
        base = c * K
        gv = jnp.full((LANES,), g, dtype=jnp.int32)
        n = _bcast_gather(n_all, gv)
        nsq_s = jnp.max(n * n)
        f_k = ((nsq_s >> 1) >> KSH) << KSH
        rho = nsq_s & 7
        paired = base + K <= f_k
        fill = base >= nsq_s
        skip = jnp.logical_and(
            jnp.logical_not(paired),
            jnp.logical_and(base >= nsq_s - f_k - rho,
                            base + K <= nsq_s - rho))
        return g, c, base, gv, n, nsq_s, rho, paired, skip, fill

    def sqrt_d_o(jj):
        a = (jj << 3) + 1
        af = a.astype(jnp.float32)
        y = lax.bitcast_convert_type(
            magic - (lax.bitcast_convert_type(af, jnp.int32) >> 1),
            jnp.float32)
        y = y * (three_half - half * af * y * y)
        y = y * (three_half - half * af * y * y)
        sq = af * y
        d0 = ((sq - 1.0) * half).astype(jnp.int32)
        d0p1 = d0 + 1
        t1 = (d0p1 * (d0 + 2)) >> 1
        up = jj >= t1
        d1 = jnp.where(up, d0p1, d0)
        tt = jnp.where(up, t1, t1 - d0p1)
        dn = jj < tt
        d = jnp.where(dn, d1 - 1, d1)
        tt = jnp.where(dn, tt - d1, tt)
        o = jj - tt
        return d, o

    def do_task(t, sbuf, rbuf, msbuf, mrbuf, sem, semM):
        g, c, base, gv, n, nsq_s, rho, paired, skip, fill = classify(t)
        nm1 = n - 1
        nsq1 = n * n - 1
        h2 = (n * n + 1) >> 1
        goff = gv << 9
        nm1g = nm1 + goff
        general = jnp.logical_not(
            jnp.logical_or(paired, jnp.logical_or(skip, fill)))
        ob = (t >> 6) * E + base

        @pl.when(paired)
        def _():
            idxr = (rho - 1 - lane) & 15
            selr = lane < rho

            def mirror_vals(o, dmo):
                return nm1g - o, nm1g - dmo

            def vec_direct(i):
                jj = jnp.full((LANES,), base + i * LANES, dtype=jnp.int32) + lane
                d, o = sqrt_d_o(jj)
                dmo = d - o
                sbuf[pl.ds(i * LANES, LANES)] = o + goff
                rbuf[pl.ds(i * LANES, LANES)] = dmo + goff
                return mirror_vals(o, dmo)

            def vec_body(i, carry):
                smp, rmp = carry
                smc, rmc = vec_direct(i)
                ri = (NVEC - i) * LANES
                msbuf[pl.ds(ri, LANES)] = jnp.where(
                    selr, _bcast_gather(smc, idxr), _bcast_gather(smp, idxr))
                mrbuf[pl.ds(ri, LANES)] = jnp.where(
                    selr, _bcast_gather(rmc, idxr), _bcast_gather(rmp, idxr))
                return smc, rmc

            sm0, rm0 = vec_direct(0)
            smL, rmL = lax.fori_loop(1, NVEC, vec_body, (sm0, rm0), unroll=4)

            jx = jnp.full((LANES,), base + K, dtype=jnp.int32) + lane
            mx = jx >= h2
            jjx = jnp.where(mx, nsq1 - jx, jx)
            dx, ox = sqrt_d_o(jjx)
            sx = jnp.where(mx, nm1 - ox, ox)
            rx = jnp.where(mx, nm1 - dx + ox, dx - ox)
            smE = nm1g - sx
            rmE = nm1g - rx
            msbuf[pl.ds(0, LANES)] = jnp.where(
                selr, _bcast_gather(smE, idxr), _bcast_gather(smL, idxr))
            mrbuf[pl.ds(0, LANES)] = jnp.where(
                selr, _bcast_gather(rmE, idxr), _bcast_gather(rmL, idxr))

            obm = (t >> 6) * E + (nsq_s >> 3) * 8 - (base + K)
            pltpu.async_copy(sbuf, send_hbm.at[pl.ds(ob, K)], sem)
            pltpu.async_copy(rbuf, recv_hbm.at[pl.ds(ob, K)], sem)
            pltpu.async_copy(msbuf, send_hbm.at[pl.ds(obm, K)], semM)
            pltpu.async_copy(mrbuf, recv_hbm.at[pl.ds(obm, K)], semM)

        @pl.when(general)
        def _():
            def vec_body(i, _):
                j = jnp.full((LANES,), base + i * LANES, dtype=jnp.int32) + lane
                m = j >= h2
                pad = j >= n * n
                jj = jnp.where(m, nsq1 - j, j)
                d, o = sqrt_d_o(jj)
                s = jnp.where(m, nm1 - o, o)
                r = jnp.where(m, nm1 - d + o, d - o)
                sbuf[pl.ds(i * LANES, LANES)] = jnp.where(pad, n, s) + goff
                rbuf[pl.ds(i * LANES, LANES)] = jnp.where(pad, n, r) + goff
                return 0

            lax.fori_loop(0, NVEC, vec_body, 0, unroll=4)
            pltpu.async_copy(sbuf, send_hbm.at[pl.ds(ob, K)], sem)
            pltpu.async_copy(rbuf, recv_hbm.at[pl.ds(ob, K)], sem)

        @pl.when(fill)
        def _():
            fillv = n + goff

            def fill_body(i, _):
                sbuf[pl.ds(i * LANES, LANES)] = fillv
                return 0

            lax.fori_loop(0, NVEC, fill_body, 0, unroll=4)
            pltpu.async_copy(sbuf, send_hbm.at[pl.ds(ob, K)], sem)
            pltpu.async_copy(sbuf, recv_hbm.at[pl.ds(ob, K)], sem)

    def wait_task(t, sbuf, rbuf, msbuf, mrbuf, sem, semM):
        _, _, _, _, _, _, _, paired, skip, _ = classify(t)

        @pl.when(jnp.logical_not(skip))
        def _():
            pltpu.make_async_copy(sbuf, send_hbm.at[pl.ds(0, K)], sem).wait()
            pltpu.make_async_copy(rbuf, recv_hbm.at[pl.ds(0, K)], sem).wait()

        @pl.when(paired)
        def _():
            pltpu.make_async_copy(msbuf, send_hbm.at[pl.ds(0, K)], semM).wait()
            pltpu.make_async_copy(mrbuf, recv_hbm.at[pl.ds(0, K)], semM).wait()

    def pair_body(k2, _):
        tA = task_id(k2)

        @pl.when(k2 > 0)
        def _():
            tP = task_id(k2 - 1)
            wait_task(tP, sbufA, rbufA, msbufA, mrbufA, semA, semMA)

        do_task(tA, sbufA, rbufA, msbufA, mrbufA, semA, semMA)

        @pl.when(k2 > 0)
        def _():
            tP = task_id(k2 - 1) + NW
            wait_task(tP, sbufB, rbufB, msbufB, mrbufB, semB, semMB)

        do_task(tA + NW, sbufB, rbufB, msbufB, mrbufB, semB, semMB)
        return 0

    lax.fori_loop(0, NPAIR, pair_body, 0)

    tL = task_id(NPAIR - 1)
    wait_task(tL, sbufA, rbufA, msbufA, mrbufA, semA, semMA)
    wait_task(tL + NW, sbufB, rbufB, msbufB, mrbufB, semB, semMB)

    @pl.when(w < 2)
    def _():
        vw = jnp.where(jnp.full((LANES,), w, dtype=jnp.int32) == 0, v0, v1)
        nv = _bcast_gather(vw, (lane >> 1) << 1)
        even = (lane & 1) == 0
        nnbuf[...] = jnp.where(even, nv, MAXN - nv)
        nebuf[...] = jnp.where(even, nv * nv, E - nv * nv)
        pltpu.sync_copy(nnbuf, nnode_hbm.at[pl.ds(w * LANES, LANES)])
        pltpu.sync_copy(nebuf, nedge_hbm.at[pl.ds(w * LANES, LANES)])


@jax.jit
def kernel(x):
    out_type = (
        jax.ShapeDtypeStruct((BATCH * E,), jnp.int32),
        jax.ShapeDtypeStruct((BATCH * E,), jnp.int32),
        jax.ShapeDtypeStruct((2 * BATCH,), jnp.int32),
        jax.ShapeDtypeStruct((2 * BATCH,), jnp.int32),
    )
    f = pl.kernel(
        _edges_kernel,
        out_type=out_type,
        mesh=_MESH,
        compiler_params=pltpu.CompilerParams(needs_layout_passes=False),
        scratch_types=[
            pltpu.VMEM((2 * BATCH,), jnp.int32),
            pltpu.VMEM((K,), jnp.int32),
            pltpu.VMEM((K,), jnp.int32),
            pltpu.VMEM((K,), jnp.int32),
            pltpu.VMEM((K,), jnp.int32),
            pltpu.VMEM((K,), jnp.int32),
            pltpu.VMEM((K,), jnp.int32),
            pltpu.VMEM((K,), jnp.int32),
            pltpu.VMEM((K,), jnp.int32),
            pltpu.VMEM((LANES,), jnp.int32),
            pltpu.VMEM((LANES,), jnp.int32),
            pltpu.SemaphoreType.DMA,
            pltpu.SemaphoreType.DMA,
            pltpu.SemaphoreType.DMA,
            pltpu.SemaphoreType.DMA,
        ],
    )
    return f(x.astype(jnp.int32).reshape(2 * BATCH))
